# baseline (speedup 1.0000x reference)
import functools

import jax
import jax.numpy as jnp
from jax import lax
from jax.experimental import pallas as pl
from jax.experimental.pallas import tpu as pltpu
from jax.experimental.pallas import tpu_sc as plsc

_VOCAB = 1000
_D = 16
_B = 16384
_F = 26

_NC = 2
_NS = 16
_NW = _NC * _NS

_FP = 32
_ROWS_W = _B // _NW
_NB = 16
_NCHUNK = _ROWS_W // _NB


def _rsqrt(x):
    i = lax.bitcast_convert_type(x, jnp.int32)
    i = jnp.int32(0x5F3759DF) - (i >> 1)
    y = lax.bitcast_convert_type(i, jnp.float32)
    for _ in range(3):
        y = y * (jnp.float32(1.5) - jnp.float32(0.5) * x * y * y)
    return y


def _body(table_hbm, idx_hbm, out_hbm, idx_v, buf, gsem):
    wid = lax.axis_index("s") * _NC + lax.axis_index("c")

    pltpu.sync_copy(idx_hbm.at[pl.ds(wid * _ROWS_W, _ROWS_W)], idx_v)

    def normalize_row(r, _):
        vs = [buf[r, f] for f in range(_F)]
        acc = vs[0] * vs[0]
        for f in range(1, _F):
            acc = acc + vs[f] * vs[f]
        norm = acc * _rsqrt(acc)
        recip = jnp.float32(1.0) / jnp.maximum(norm, jnp.float32(1e-12))
        for f in range(_F):
            buf[r, f] = vs[f] * recip
        return _

    def chunk(c, _):
        row0 = c * _NB
        copies = []
        for r in range(_NB):
            copies.append(pltpu.async_copy(
                table_hbm.at[idx_v.at[row0 + r, pl.ds(0, _FP)]],
                buf.at[r], gsem))
        for cp in copies:
            cp.wait()
        lax.fori_loop(0, _NB, normalize_row, None)
        pltpu.sync_copy(buf.at[:, pl.ds(0, _F)],
                        out_hbm.at[pl.ds(wid * _ROWS_W + row0, _NB)])
        return _

    lax.fori_loop(0, _NCHUNK, chunk, None)


def kernel(user_features, embedding_table):
    mesh = plsc.VectorSubcoreMesh(
        core_axis_name="c", subcore_axis_name="s",
        num_cores=_NC, num_subcores=_NS)
    run = functools.partial(
        pl.kernel,
        out_type=jax.ShapeDtypeStruct((_B, _F, _D), jnp.float32),
        mesh=mesh,
        scratch_types=[
            pltpu.VMEM((_ROWS_W, 128), jnp.int32),
            pltpu.VMEM((_NB, _FP, _D), jnp.float32),
            pltpu.SemaphoreType.DMA,
        ],
        compiler_params=pltpu.CompilerParams(use_tc_tiling_on_sc=False),
    )(_body)
    idx_pad = jnp.pad(user_features, ((0, 0), (0, 128 - _F)))
    return run(embedding_table, idx_pad)


# speedup vs baseline: 6.1717x; 6.1717x over previous
import functools

import jax
import jax.numpy as jnp
from jax import lax
from jax.experimental import pallas as pl
from jax.experimental.pallas import tpu as pltpu
from jax.experimental.pallas import tpu_sc as plsc

_VOCAB = 1000
_D = 16
_B = 16384
_F = 26

_NC = 2
_NS = 16
_NW = _NC * _NS

_BBLK = 128
_NBLK = _B // _BBLK
_BLK_W = _NBLK // _NW
_LANES = 16


def _recip_norm(acc):
    i = lax.bitcast_convert_type(acc, jnp.int32)
    i = jnp.int32(0x5F3759DF) - (i >> 1)
    y = lax.bitcast_convert_type(i, jnp.float32)
    for _ in range(3):
        y = y * (jnp.float32(1.5) - jnp.float32(0.5) * acc * y * y)
    norm = acc * y
    return jnp.float32(1.0) / jnp.maximum(norm, jnp.float32(1e-12))


def _body(table_hbm, idx_hbm, out_hbm, table_v, idx_v, out_st, gsem):
    wid = lax.axis_index("s") * _NC + lax.axis_index("c")

    pltpu.sync_copy(table_hbm, table_v)

    def do_block(cb, _):
        blk = wid * _BLK_W + cb
        pltpu.sync_copy(idx_hbm.at[:, pl.ds(blk * _BBLK, _BBLK)], idx_v)

        def do_group(bb, _):
            fv = [idx_v[f, pl.ds(bb * _LANES, _LANES)] * _D
                  for f in range(_F)]

            def do_dim(d, _):
                gs = [plsc.load_gather(table_v, [fv[f] + d])
                      for f in range(_F)]
                acc = gs[0] * gs[0]
                for f in range(1, _F):
                    acc = acc + gs[f] * gs[f]
                recip = _recip_norm(acc)
                dhi = d >> 3
                dlo = d & 7
                for f in range(_F):
                    out_st[f, dhi, dlo, pl.ds(bb * _LANES, _LANES)] = (
                        gs[f] * recip)
                return _

            lax.fori_loop(0, _D, do_dim, None)
            return _

        lax.fori_loop(0, _BBLK // _LANES, do_group, None)
        pltpu.sync_copy(out_st, out_hbm.at[:, :, blk])
        return _

    lax.fori_loop(0, _BLK_W, do_block, None)


def kernel(user_features, embedding_table):
    mesh = plsc.VectorSubcoreMesh(
        core_axis_name="c", subcore_axis_name="s",
        num_cores=_NC, num_subcores=_NS)
    run = functools.partial(
        pl.kernel,
        out_type=jax.ShapeDtypeStruct((_F, _D // 8, _NBLK, 8, _BBLK),
                                      jnp.float32),
        mesh=mesh,
        scratch_types=[
            pltpu.VMEM((_VOCAB * _D,), jnp.float32),
            pltpu.VMEM((_F, _BBLK), jnp.int32),
            pltpu.VMEM((_F, _D // 8, 8, _BBLK), jnp.float32),
            pltpu.SemaphoreType.DMA,
        ],
        compiler_params=pltpu.CompilerParams(use_tc_tiling_on_sc=False, needs_layout_passes=False),
    )(_body)
    idx_t = user_features.T
    out5 = run(embedding_table.reshape(_VOCAB * _D), idx_t)
    out = jnp.transpose(out5, (2, 4, 0, 1, 3))
    return out.reshape(_B, _F, _D)
